# baseline (speedup 1.0000x reference)
import functools

import jax
import jax.numpy as jnp
from jax import lax
from jax.experimental import pallas as pl
from jax.experimental.pallas import tpu as pltpu
from jax.experimental.pallas import tpu_sc as plsc
from jax.experimental import layout as jax_layout

NC = 2
NS = 16
NW = NC * NS
LANES = 16

B = 4096
L = 200
D = 64
N = B * L
SEQ_PER_CHUNK = 2
R = SEQ_PER_CHUNK * L
SUBSLICES = ((0, 80), (80, 80), (160, 40))
SEQ_PER_W = B // NW
CHUNKS = SEQ_PER_W // SEQ_PER_CHUNK


def _body(x_hbm, tok_hbm, pos_hbm, out_hbm, idx_v, rows_v, pos_v, sem):
    cid = lax.axis_index("c")
    sid = lax.axis_index("s")
    wid = sid * NC + cid

    pltpu.sync_copy(pos_hbm, pos_v)

    def chunk_body(c, carry):
        seq0 = wid * SEQ_PER_W + c * SEQ_PER_CHUNK

        pltpu.sync_copy(x_hbm.at[pl.ds(seq0, SEQ_PER_CHUNK)], idx_v)

        copies = []
        for s in range(SEQ_PER_CHUNK):
            for o, w in SUBSLICES:
                copies.append(
                    pltpu.async_copy(
                        tok_hbm.at[idx_v.at[s, pl.ds(o, w)]],
                        rows_v.at[s, pl.ds(o, w)],
                        sem,
                    )
                )
        for cp in copies:
            cp.wait()

        def add_body(l, carry2):
            for j in range(D // LANES):
                pv = pos_v[l, pl.ds(j * LANES, LANES)]
                for s in range(SEQ_PER_CHUNK):
                    rows_v[s, l, pl.ds(j * LANES, LANES)] = (
                        rows_v[s, l, pl.ds(j * LANES, LANES)] + pv
                    )
            return carry2

        lax.fori_loop(0, L, add_body, 0)

        pltpu.sync_copy(rows_v, out_hbm.at[pl.ds(seq0, SEQ_PER_CHUNK)])
        return carry

    lax.fori_loop(0, CHUNKS, chunk_body, 0)


def kernel(x, token_table, pos_table):
    mesh = plsc.VectorSubcoreMesh(core_axis_name="c", subcore_axis_name="s")
    out = pl.kernel(
        _body,
        mesh=mesh,
        out_type=jax.ShapeDtypeStruct((B, L, D), jnp.float32),
        compiler_params=pltpu.CompilerParams(use_tc_tiling_on_sc=False),
        scratch_types=[
            pltpu.VMEM((SEQ_PER_CHUNK, L), jnp.int32),
            pltpu.VMEM((SEQ_PER_CHUNK, L, D), jnp.float32),
            pltpu.VMEM((L, D), jnp.float32),
            pltpu.SemaphoreType.DMA,
        ],
    )(x.astype(jnp.int32), token_table, pos_table)
    return out


kernel = jax.jit(kernel, out_shardings=jax_layout.Format(jax_layout.Layout.AUTO))


# speedup vs baseline: 1.0045x; 1.0045x over previous
import functools

import jax
import jax.numpy as jnp
from jax import lax
from jax.experimental import pallas as pl
from jax.experimental.pallas import tpu as pltpu
from jax.experimental.pallas import tpu_sc as plsc
from jax.experimental import layout as jax_layout

NC = 2
NS = 16
NW = NC * NS
LANES = 16

B = 4096
L = 200
D = 64
N = B * L
SEQ_PER_CHUNK = 2
R = SEQ_PER_CHUNK * L
SUBSLICES = ((0, 80), (80, 80), (160, 40))
SEQ_PER_W = B // NW
CHUNKS = SEQ_PER_W // SEQ_PER_CHUNK


def _body(x_hbm, tok_hbm, pos_hbm, out_hbm, idx_v, rows_v, pos_v, sem):
    cid = lax.axis_index("c")
    sid = lax.axis_index("s")
    wid = sid * NC + cid

    pltpu.sync_copy(pos_hbm, pos_v)

    def chunk_body(c, carry):
        seq0 = wid * SEQ_PER_W + c * SEQ_PER_CHUNK

        pltpu.sync_copy(x_hbm.at[pl.ds(seq0, SEQ_PER_CHUNK)], idx_v)

        copies = []
        for s in range(SEQ_PER_CHUNK):
            for o, w in SUBSLICES:
                copies.append(
                    pltpu.async_copy(
                        tok_hbm.at[idx_v.at[s, pl.ds(o, w)]],
                        rows_v.at[s, pl.ds(o, w)],
                        sem,
                    )
                )
        for cp in copies:
            cp.wait()

        def add_body(l, carry2):
            for j in range(D // LANES):
                pv = pos_v[l, pl.ds(j * LANES, LANES)]
                for s in range(SEQ_PER_CHUNK):
                    rows_v[s, l, pl.ds(j * LANES, LANES)] = (
                        rows_v[s, l, pl.ds(j * LANES, LANES)] + pv
                    )
            return carry2

        lax.fori_loop(0, L, add_body, 0)

        pltpu.sync_copy(rows_v, out_hbm.at[pl.ds(seq0, SEQ_PER_CHUNK)])
        return carry

    lax.fori_loop(0, CHUNKS, chunk_body, 0)


def _kernel_impl(x, token_table, pos_table):
    mesh = plsc.VectorSubcoreMesh(core_axis_name="c", subcore_axis_name="s")
    out = pl.kernel(
        _body,
        mesh=mesh,
        out_type=jax.ShapeDtypeStruct((B, L, D), jnp.float32),
        compiler_params=pltpu.CompilerParams(use_tc_tiling_on_sc=False),
        scratch_types=[
            pltpu.VMEM((SEQ_PER_CHUNK, L), jnp.int32),
            pltpu.VMEM((SEQ_PER_CHUNK, L, D), jnp.float32),
            pltpu.VMEM((L, D), jnp.float32),
            pltpu.SemaphoreType.DMA,
        ],
    )(x.astype(jnp.int32), token_table, pos_table)
    return out


_kernel_impl.__name__ = "kernel"
_jitted = None


def kernel(x, token_table, pos_table):
    global _jitted
    if _jitted is None:
        dev = jax.devices()[0]
        fmt = jax_layout.Format(
            jax_layout.Layout(major_to_minor=(0, 1, 2), tiling=()),
            jax.sharding.SingleDeviceSharding(dev),
        )
        _jitted = jax.jit(_kernel_impl, out_shardings=fmt)
    return _jitted(x, token_table, pos_table)
